# baseline (speedup 1.0000x reference)
import functools

import jax
import jax.numpy as jnp
from jax import lax
from jax.experimental import pallas as pl
from jax.experimental.pallas import tpu as pltpu
from jax.experimental.pallas import tpu_sc as plsc

NUM_EMBEDDINGS = 1000000
FEATURES = 32
BATCH = 16384
HIST = 50

_B = BATCH * HIST
_NW = 32
_B_PER_W = _B // _NW
_BR_PER_W = _B_PER_W // HIST
_STEP_BR = 8
_STEP = _STEP_BR * HIST
_NBUF = 4
_NSTEPS = _BR_PER_W // _STEP_BR
assert _NSTEPS % _NBUF == 0


def _make_gather():
    mesh = plsc.VectorSubcoreMesh(core_axis_name="c", subcore_axis_name="s")

    @functools.partial(
        pl.kernel,
        mesh=mesh,
        out_type=jax.ShapeDtypeStruct((BATCH, HIST, FEATURES), jnp.float32),
        scratch_types=[
            pltpu.VMEM((_B_PER_W,), jnp.int32),
            pltpu.VMEM((_NBUF, _STEP, FEATURES), jnp.float32),
            pltpu.SemaphoreType.DMA((_NBUF,)),
            pltpu.SemaphoreType.DMA((_NBUF,)),
        ],
        compiler_params=pltpu.CompilerParams(use_tc_tiling_on_sc=False),
    )
    def k(table_hbm, idx_hbm, out_hbm, idx_v, rows_v, gsem, osem):
        wid = lax.axis_index("s") * 2 + lax.axis_index("c")
        base = wid * _B_PER_W
        base_br = wid * _BR_PER_W
        pltpu.sync_copy(idx_hbm.at[pl.ds(base, _B_PER_W)], idx_v)

        def start_gather(g, b):
            pltpu.async_copy(
                table_hbm.at[idx_v.at[pl.ds(g * _STEP, _STEP)]],
                rows_v.at[b],
                gsem.at[b],
            )

        def wait_gather(g, b):
            pltpu.make_async_copy(
                table_hbm.at[idx_v.at[pl.ds(g * _STEP, _STEP)]],
                rows_v.at[b],
                gsem.at[b],
            ).wait()

        def start_out(g, b):
            for r in range(_STEP_BR):
                pltpu.async_copy(
                    rows_v.at[b].at[pl.ds(r * HIST, HIST)],
                    out_hbm.at[base_br + g * _STEP_BR + r],
                    osem.at[b],
                )

        def drain_out(g, b):
            for r in range(_STEP_BR):
                pltpu.make_async_copy(
                    rows_v.at[b].at[pl.ds(r * HIST, HIST)],
                    out_hbm.at[base_br + g * _STEP_BR + r],
                    osem.at[b],
                ).wait()

        for b in range(_NBUF):
            start_gather(b, b)

        def outer(o, carry):
            g0 = o * _NBUF
            for b in range(_NBUF):
                g = g0 + b
                wait_gather(g, b)
                start_out(g, b)
                drain_out(g, b)
                start_gather(g + _NBUF, b)
            return carry

        lax.fori_loop(0, _NSTEPS // _NBUF - 1, outer, 0)
        g0 = _NSTEPS - _NBUF
        for b in range(_NBUF):
            wait_gather(g0 + b, b)
            start_out(g0 + b, b)
            drain_out(g0 + b, b)

    return k


_gather = _make_gather()


def kernel(inputs, embedding):
    idx = jnp.asarray(inputs, jnp.int32).reshape(-1)
    table = jnp.asarray(embedding, jnp.float32)
    return _gather(table, idx)


# speedup vs baseline: 1.0008x; 1.0008x over previous
import functools

import jax
import jax.numpy as jnp
from jax import lax
from jax.experimental import pallas as pl
from jax.experimental.pallas import tpu as pltpu
from jax.experimental.pallas import tpu_sc as plsc

NUM_EMBEDDINGS = 1000000
FEATURES = 32
BATCH = 16384
HIST = 50

_B = BATCH * HIST
_NW = 32
_B_PER_W = _B // _NW
_BR_PER_W = _B_PER_W // HIST
_STEP_BR = 8
_STEP = _STEP_BR * HIST
_NBUF = 4
_NSTEPS = _BR_PER_W // _STEP_BR
assert _NSTEPS % _NBUF == 0


def _make_gather():
    mesh = plsc.VectorSubcoreMesh(core_axis_name="c", subcore_axis_name="s")

    @functools.partial(
        pl.kernel,
        mesh=mesh,
        out_type=jax.ShapeDtypeStruct((BATCH, HIST, FEATURES), jnp.float32),
        scratch_types=[
            pltpu.VMEM((_B_PER_W,), jnp.int32),
            pltpu.VMEM((_NBUF, _STEP, FEATURES), jnp.float32),
            pltpu.SemaphoreType.DMA((_NBUF,)),
            pltpu.SemaphoreType.DMA((_NBUF,)),
        ],
        compiler_params=pltpu.CompilerParams(use_tc_tiling_on_sc=False),
    )
    def k(table_hbm, idx_hbm, out_hbm, idx_v, rows_v, gsem, osem):
        wid = lax.axis_index("s") * 2 + lax.axis_index("c")
        base = wid * _B_PER_W
        base_br = wid * _BR_PER_W
        pltpu.sync_copy(idx_hbm.at[pl.ds(base, _B_PER_W)], idx_v)

        def start_gather(g, b):
            pltpu.async_copy(
                table_hbm.at[idx_v.at[pl.ds(g * _STEP, _STEP)]],
                rows_v.at[b],
                gsem.at[b],
            )

        def wait_gather(g, b):
            pltpu.make_async_copy(
                table_hbm.at[idx_v.at[pl.ds(g * _STEP, _STEP)]],
                rows_v.at[b],
                gsem.at[b],
            ).wait()

        def start_out(g, b):
            for r in range(_STEP_BR):
                pltpu.async_copy(
                    rows_v.at[b].at[pl.ds(r * HIST, HIST)],
                    out_hbm.at[base_br + g * _STEP_BR + r],
                    osem.at[b],
                )

        def drain_out(g, b):
            for r in range(_STEP_BR):
                pltpu.make_async_copy(
                    rows_v.at[b].at[pl.ds(r * HIST, HIST)],
                    out_hbm.at[base_br + g * _STEP_BR + r],
                    osem.at[b],
                ).wait()

        for b in range(_NBUF):
            start_gather(b, b)

        def outer(o, carry):
            g0 = o * _NBUF
            for b in range(_NBUF):
                g = g0 + b
                wait_gather(g, b)
                start_out(g, b)
                pb = (b - 1) % _NBUF
                gp = g - 1

                @pl.when(jnp.logical_and(gp >= 0, gp + _NBUF < _NSTEPS))
                def _():
                    drain_out(gp, pb)
                    start_gather(gp + _NBUF, pb)

            return carry

        lax.fori_loop(0, _NSTEPS // _NBUF, outer, 0)
        for g in range(_NSTEPS - _NBUF, _NSTEPS):
            drain_out(g, g % _NBUF)

    return k


_gather = _make_gather()


def kernel(inputs, embedding):
    idx = jnp.asarray(inputs, jnp.int32).reshape(-1)
    table = jnp.asarray(embedding, jnp.float32)
    return _gather(table, idx)
